# baseline (speedup 1.0000x reference)
import functools

import jax
import jax.numpy as jnp
from jax.experimental import pallas as pl

GT_POS = 0.01
GT_NEG = 0.02
UNMATCHED = -1
IGNORE = -2

N = 2048
CH_A = 512
CH_B = 256


def _reproject(kpts, T):
    B, n, _ = kpts.shape
    h = jnp.concatenate([kpts, jnp.ones((B, n, 1), kpts.dtype)], axis=-1)
    h = jnp.einsum('bij,bnj->bni', T, h)
    z = h[..., 2]
    zs = jnp.where(jnp.abs(z) < 1e-8, 1e-8, z)
    pts = h[..., :2] / zs[..., None]
    mask = (z > 1e-8) & (pts[..., 0] >= 0.0) & (pts[..., 0] <= 1.0) \
           & (pts[..., 1] >= 0.0) & (pts[..., 1] <= 1.0)
    return pts, mask


def _dist_kernel(qx_ref, qy_ref, kx_ref, ky_ref, md_ref, nn_ref):
    kx = kx_ref[0]
    ky = ky_ref[0]
    jrow = jax.lax.broadcasted_iota(jnp.int32, (1, N), 1)
    for c in range(N // CH_A):
        sl = pl.ds(c * CH_A, CH_A)
        qx = qx_ref[0, sl, :]
        qy = qy_ref[0, sl, :]
        dx = qx - kx
        dy = qy - ky
        d = jnp.sqrt(dx * dx + dy * dy + 1e-12)
        mind = jnp.min(d, axis=1, keepdims=True)
        idx = jnp.min(jnp.where(d == mind, jrow, N),
                      axis=1, keepdims=True)
        md_ref[0, sl, :] = mind
        nn_ref[0, sl, :] = idx


def _cross_kernel(idx_ref, mdq_ref, mkq_ref, nnp_ref, mdp_ref, mkp_ref, gt_ref):
    nnp = nnp_ref[0]
    mdp = mdp_ref[0]
    mkp = mkp_ref[0]
    jrow = jax.lax.broadcasted_iota(jnp.int32, (1, N), 1)
    for c in range(N // CH_B):
        sl = pl.ds(c * CH_B, CH_B)
        idxc = idx_ref[0, sl, :]
        m = idxc == jrow
        g_nn = jnp.sum(jnp.where(m, nnp, 0), axis=1, keepdims=True)
        g_md = jnp.sum(jnp.where(m, mdp, 0.0), axis=1, keepdims=True)
        g_mk = jnp.sum(jnp.where(m, mkp, 0.0), axis=1, keepdims=True)
        qi = c * CH_B + jax.lax.broadcasted_iota(jnp.int32, (CH_B, 1), 0)
        cc = g_nn == qi
        sym = 0.5 * (mdq_ref[0, sl, :] + g_md)
        gt = jnp.where(cc, idxc, UNMATCHED)
        gt = jnp.where(cc & (sym > GT_POS), IGNORE, gt)
        gt = jnp.where(cc & (sym > GT_NEG), UNMATCHED, gt)
        gt = jnp.where(mkq_ref[0, sl, :] > 0.5, gt, IGNORE)
        gt = jnp.where(g_mk > 0.5, gt, IGNORE)
        gt_ref[0, sl, :] = gt


def _col_spec():
    return pl.BlockSpec((1, N, 1), lambda r: (r, 0, 0))


def _row_spec(partner=False):
    if partner:
        return pl.BlockSpec((1, 1, N), lambda r: ((r + 4) % 8, 0, 0))
    return pl.BlockSpec((1, 1, N), lambda r: (r, 0, 0))


@functools.partial(jax.jit, static_argnames=())
def kernel(kpts0, kpts1, desc0, desc1, scores0, scores1, transformation):
    T = transformation
    T_inv = jnp.linalg.inv(T)

    k0t, mask0 = _reproject(kpts0, T)
    k1t, mask1 = _reproject(kpts1, T_inv)

    qx = jnp.concatenate([k0t[..., 0], k1t[..., 0]])[..., None]
    qy = jnp.concatenate([k0t[..., 1], k1t[..., 1]])[..., None]
    kx = jnp.concatenate([kpts1[..., 0], kpts0[..., 0]])[:, None, :]
    ky = jnp.concatenate([kpts1[..., 1], kpts0[..., 1]])[:, None, :]
    maskq = jnp.concatenate([mask0, mask1]).astype(jnp.float32)

    md, nn = pl.pallas_call(
        _dist_kernel,
        grid=(8,),
        in_specs=[_col_spec(), _col_spec(), _row_spec(), _row_spec()],
        out_specs=[_col_spec(), _col_spec()],
        out_shape=[jax.ShapeDtypeStruct((8, N, 1), jnp.float32),
                   jax.ShapeDtypeStruct((8, N, 1), jnp.int32)],
    )(qx, qy, kx, ky)

    md_row = jnp.transpose(md, (0, 2, 1))
    nn_row = jnp.transpose(nn, (0, 2, 1))
    mk_row = maskq[:, None, :]
    mk_col = maskq[..., None]

    gt = pl.pallas_call(
        _cross_kernel,
        grid=(8,),
        in_specs=[_col_spec(), _col_spec(), _col_spec(),
                  _row_spec(partner=True), _row_spec(partner=True),
                  _row_spec(partner=True)],
        out_specs=_col_spec(),
        out_shape=jax.ShapeDtypeStruct((8, N, 1), jnp.int32),
    )(nn, md, mk_col, nn_row, md_row, mk_row)

    gt = gt[..., 0]
    gt0, gt1 = gt[:4], gt[4:]
    return (kpts0, kpts1, desc0, desc1, scores0, scores1, gt0, gt1)
